# baseline (speedup 1.0000x reference)
import functools

import jax
import jax.numpy as jnp
from jax import lax
from jax.experimental import pallas as pl
from jax.experimental.pallas import tpu as pltpu
from jax.experimental.pallas import tpu_sc as plsc

NUM_ROWS = 1000001
EMBED_DIM = 64
BATCH = 16384

_NC = 2
_NS = 16
_NW = _NC * _NS
_B_PER_W = BATCH // _NW

_mesh = plsc.VectorSubcoreMesh(core_axis_name="c", subcore_axis_name="s")


@functools.partial(
    pl.kernel,
    mesh=_mesh,
    out_type=jax.ShapeDtypeStruct((BATCH, EMBED_DIM), jnp.float32),
    scratch_types=[
        pltpu.VMEM((_B_PER_W,), jnp.int32),
        pltpu.VMEM((_B_PER_W, EMBED_DIM), jnp.float32),
        pltpu.SemaphoreType.DMA,
    ],
    compiler_params=pltpu.CompilerParams(use_tc_tiling_on_sc=False),
)
def _embedding_gather(idx_hbm, table_hbm, out_hbm, idx_v, rows_v, sem):
    wid = lax.axis_index("s") * _NC + lax.axis_index("c")
    base = wid * _B_PER_W
    pltpu.sync_copy(idx_hbm.at[pl.ds(base, _B_PER_W)], idx_v)
    pltpu.async_copy(table_hbm.at[idx_v], rows_v, sem).wait()
    pltpu.sync_copy(rows_v, out_hbm.at[pl.ds(base, _B_PER_W)])


def kernel(domain_ids, table):
    return _embedding_gather(domain_ids.astype(jnp.int32), table)


# speedup vs baseline: 1.7318x; 1.7318x over previous
import functools

import jax
import jax.numpy as jnp
from jax import lax
from jax.experimental import pallas as pl
from jax.experimental.pallas import tpu as pltpu
from jax.experimental.pallas import tpu_sc as plsc

NUM_ROWS = 1000001
EMBED_DIM = 64
BATCH = 16384

_NC = 2
_NS = 16
_NW = _NC * _NS
_B_PER_W = BATCH // _NW

_mesh = plsc.VectorSubcoreMesh(core_axis_name="c", subcore_axis_name="s")


@functools.partial(
    pl.kernel,
    mesh=_mesh,
    out_type=jax.ShapeDtypeStruct((BATCH, EMBED_DIM), jnp.float32),
    scratch_types=[
        pltpu.VMEM((_B_PER_W,), jnp.int32),
        pltpu.VMEM((_B_PER_W, EMBED_DIM), jnp.float32),
        pltpu.SemaphoreType.DMA,
    ],
)
def _embedding_gather(idx_hbm, table_hbm, out_hbm, idx_v, rows_v, sem_r):
    wid = lax.axis_index("s") * _NC + lax.axis_index("c")
    base = wid * _B_PER_W
    pltpu.sync_copy(idx_hbm.at[pl.ds(base, _B_PER_W)], idx_v)

    def body(g, carry):
        vec = idx_v[pl.ds(g * 16, 16)]
        for k in range(16):
            idx = vec[k]
            pltpu.async_copy(table_hbm.at[pl.ds(idx, 1)],
                             rows_v.at[pl.ds(g * 16 + k, 1)], sem_r)
        return carry

    lax.fori_loop(0, _B_PER_W // 16, body, 0)
    pltpu.make_async_copy(table_hbm.at[pl.ds(0, _B_PER_W)], rows_v,
                          sem_r).wait()
    pltpu.sync_copy(rows_v, out_hbm.at[pl.ds(base, _B_PER_W)])


def kernel(domain_ids, table):
    return _embedding_gather(domain_ids.astype(jnp.int32), table)


# speedup vs baseline: 2.6812x; 1.5482x over previous
import functools

import jax
import jax.numpy as jnp
from jax import lax
from jax.experimental import pallas as pl
from jax.experimental.pallas import tpu as pltpu
from jax.experimental.pallas import tpu_sc as plsc

NUM_ROWS = 1000001
EMBED_DIM = 64
BATCH = 16384

_NC = 2
_NS = 16
_NW = _NC * _NS
_B_PER_W = BATCH // _NW
_TAIL_START = (NUM_ROWS - 1) // 128 * 128
_TAIL_LEN = NUM_ROWS - _TAIL_START

_mesh = plsc.VectorSubcoreMesh(core_axis_name="c", subcore_axis_name="s")


@functools.partial(
    pl.kernel,
    mesh=_mesh,
    out_type=jax.ShapeDtypeStruct((BATCH * EMBED_DIM,), jnp.float32),
    scratch_types=[
        pltpu.VMEM((_B_PER_W,), jnp.int32),
        pltpu.VMEM((8, EMBED_DIM, 128), jnp.float32),
        pltpu.VMEM((_B_PER_W * EMBED_DIM,), jnp.float32),
        pltpu.VMEM((1, EMBED_DIM), jnp.float32),
        pltpu.SemaphoreType.DMA,
    ],
    compiler_params=pltpu.CompilerParams(needs_layout_passes=False),
)
def _embedding_gather(idx_hbm, tt_hbm, tail_hbm, out_hbm,
                      idx_v, tcb, rowbuf, tailbuf, sem):
    wid = lax.axis_index("s") * _NC + lax.axis_index("c")
    base = wid * _B_PER_W
    pltpu.sync_copy(idx_hbm.at[pl.ds(base, _B_PER_W)], idx_v)

    def fire(vec, lane0, slot0):
        for j in range(4):
            idx = vec[lane0 + j]
            tcol = jnp.where(idx >= _TAIL_START, 0, idx >> 7)
            start = pl.multiple_of(tcol * 128, 128)
            pltpu.async_copy(tt_hbm.at[:, pl.ds(start, 128)],
                             tcb.at[slot0 + j], sem)

    def drain(slot0):
        for j in range(4):
            pltpu.make_async_copy(tt_hbm.at[:, pl.ds(0, 128)],
                                  tcb.at[slot0 + j], sem).wait()

    def extract(q, vec, w, slot0):
        for j in range(4):
            idx = vec[4 * w + j]
            rm = lax.broadcast(idx & 127, (16,))
            gid64 = (q * 16 + 4 * w + j) * EMBED_DIM

            @pl.when(idx < _TAIL_START)
            def _():
                for c in range(4):
                    cvec = lax.iota(jnp.int32, 16) + 16 * c
                    vals = plsc.load_gather(tcb.at[slot0 + j], [cvec, rm])
                    rowbuf[pl.ds(gid64 + 16 * c, 16)] = vals

            @pl.when(idx >= _TAIL_START)
            def _():
                pltpu.sync_copy(tail_hbm.at[pl.ds(idx - _TAIL_START, 1)],
                                tailbuf)
                for c in range(4):
                    rowbuf[pl.ds(gid64 + 16 * c, 16)] = (
                        tailbuf.at[0][pl.ds(16 * c, 16)])

    def body(q, carry):
        vec = idx_v[pl.ds(q * 16, 16)]
        fire(vec, 0, 0)
        fire(vec, 4, 4)
        drain(0)
        extract(q, vec, 0, 0)
        fire(vec, 8, 0)
        drain(4)
        extract(q, vec, 1, 4)
        fire(vec, 12, 4)
        drain(0)
        extract(q, vec, 2, 0)
        drain(4)
        extract(q, vec, 3, 4)
        return carry

    lax.fori_loop(0, _B_PER_W // 16, body, 0)
    pltpu.sync_copy(rowbuf, out_hbm.at[pl.ds(base * EMBED_DIM,
                                              _B_PER_W * EMBED_DIM)])


def kernel(domain_ids, table):
    tail = lax.slice(table, (_TAIL_START, 0), (NUM_ROWS, EMBED_DIM))
    out_flat = _embedding_gather(domain_ids.astype(jnp.int32), table.T, tail)
    return out_flat.reshape(BATCH, EMBED_DIM)


# speedup vs baseline: 2.7528x; 1.0267x over previous
import functools

import jax
import jax.numpy as jnp
from jax import lax
from jax.experimental import pallas as pl
from jax.experimental.pallas import tpu as pltpu
from jax.experimental.pallas import tpu_sc as plsc

NUM_ROWS = 1000001
EMBED_DIM = 64
BATCH = 16384

_NC = 2
_NS = 16
_NW = _NC * _NS
_B_PER_W = BATCH // _NW
_TAIL_START = (NUM_ROWS - 1) // 128 * 128
_TAIL_LEN = NUM_ROWS - _TAIL_START

_mesh = plsc.VectorSubcoreMesh(core_axis_name="c", subcore_axis_name="s")


@functools.partial(
    pl.kernel,
    mesh=_mesh,
    out_type=jax.ShapeDtypeStruct((BATCH * EMBED_DIM,), jnp.float32),
    scratch_types=[
        pltpu.VMEM((_B_PER_W + 16,), jnp.int32),
        pltpu.VMEM((8, EMBED_DIM, 128), jnp.float32),
        pltpu.VMEM((_B_PER_W * EMBED_DIM,), jnp.float32),
        pltpu.VMEM((1, EMBED_DIM), jnp.float32),
        pltpu.SemaphoreType.DMA,
    ],
    compiler_params=pltpu.CompilerParams(needs_layout_passes=False),
)
def _embedding_gather(idx_hbm, tt_hbm, tail_hbm, out_hbm,
                      idx_v, tcb, rowbuf, tailbuf, sem):
    wid = lax.axis_index("s") * _NC + lax.axis_index("c")
    base = wid * _B_PER_W
    pltpu.sync_copy(idx_hbm.at[pl.ds(base, _B_PER_W)],
                    idx_v.at[pl.ds(0, _B_PER_W)])

    def fire(vec, lane0, slot0):
        for j in range(4):
            idx = vec[lane0 + j]
            tcol = jnp.where(idx >= _TAIL_START, 0, idx >> 7)
            tcol = jnp.clip(tcol, 0, (_TAIL_START // 128) - 1)
            start = pl.multiple_of(tcol * 128, 128)
            pltpu.async_copy(tt_hbm.at[:, pl.ds(start, 128)],
                             tcb.at[slot0 + j], sem)

    def drain(slot0):
        for j in range(4):
            pltpu.make_async_copy(tt_hbm.at[:, pl.ds(0, 128)],
                                  tcb.at[slot0 + j], sem).wait()

    def extract(q, vec, w, slot0):
        for j in range(4):
            idx = vec[4 * w + j]
            rm = lax.broadcast(idx & 127, (16,))
            gid64 = (q * 16 + 4 * w + j) * EMBED_DIM

            @pl.when(idx < _TAIL_START)
            def _():
                for c in range(4):
                    cvec = lax.iota(jnp.int32, 16) + 16 * c
                    vals = plsc.load_gather(tcb.at[slot0 + j], [cvec, rm])
                    rowbuf[pl.ds(gid64 + 16 * c, 16)] = vals

            @pl.when(idx >= _TAIL_START)
            def _():
                pltpu.sync_copy(tail_hbm.at[pl.ds(idx - _TAIL_START, 1)],
                                tailbuf)
                for c in range(4):
                    rowbuf[pl.ds(gid64 + 16 * c, 16)] = (
                        tailbuf.at[0][pl.ds(16 * c, 16)])

    def body(q, carry):
        vec = idx_v[pl.ds(q * 16, 16)]
        vec_n = idx_v[pl.ds(q * 16 + 16, 16)]
        drain(0)
        extract(q, vec, 0, 0)
        fire(vec, 8, 0)
        drain(4)
        extract(q, vec, 1, 4)
        fire(vec, 12, 4)
        drain(0)
        extract(q, vec, 2, 0)
        fire(vec_n, 0, 0)
        drain(4)
        extract(q, vec, 3, 4)
        fire(vec_n, 4, 4)
        return carry

    vec0 = idx_v[pl.ds(0, 16)]
    fire(vec0, 0, 0)
    fire(vec0, 4, 4)
    lax.fori_loop(0, _B_PER_W // 16, body, 0)
    drain(0)
    drain(4)
    pltpu.sync_copy(rowbuf, out_hbm.at[pl.ds(base * EMBED_DIM,
                                              _B_PER_W * EMBED_DIM)])


def kernel(domain_ids, table):
    tail = lax.slice(table, (_TAIL_START, 0), (NUM_ROWS, EMBED_DIM))
    out_flat = _embedding_gather(domain_ids.astype(jnp.int32), table.T, tail)
    return out_flat.reshape(BATCH, EMBED_DIM)
